# Initial kernel scaffold; baseline (speedup 1.0000x reference)
import jax
import jax.numpy as jnp
from jax.experimental import pallas as pl


def kernel(pred, target):
    raise NotImplementedError("write your pallas kernel here")


# speedup vs baseline: 4.5106x; 4.5106x over previous
import jax
import jax.numpy as jnp
from jax.experimental import pallas as pl
from jax.experimental.pallas import tpu as pltpu

V = 32000
SMOOTHING = 0.1
IGNORE = 0
CONF = 1.0 - SMOOTHING
SMOOTH = SMOOTHING / (V - 1)

BR = 16


def _body(t_ref, p_ref, loss_ref, cnt_ref):
    i = pl.program_id(0)
    p = p_ref[...]
    t = t_ref[0, 0, :]
    m = jnp.max(p, axis=1, keepdims=True)
    s = jnp.sum(jnp.exp(p - m), axis=1)
    lse = m[:, 0] + jnp.log(s)
    psum = jnp.sum(p, axis=1)
    col = jax.lax.broadcasted_iota(jnp.int32, (BR, V), 1)
    pt = jnp.sum(jnp.where(col == t[:, None], p, 0.0), axis=1)
    maskf = (t != IGNORE).astype(jnp.float32)
    loss = -(SMOOTH * (psum - V * lse) + (CONF - SMOOTH) * (pt - lse))

    @pl.when(i == 0)
    def _():
        loss_ref[0, 0] = 0.0
        cnt_ref[0, 0] = 0.0

    loss_ref[0, 0] += jnp.sum(loss * maskf)
    cnt_ref[0, 0] += jnp.sum(maskf)


def kernel(pred, target):
    p = pred.reshape(-1, V)
    n = p.shape[0]
    nb = n // BR
    t3 = target.reshape(nb, 1, BR).astype(jnp.int32)

    loss_sum, cnt = pl.pallas_call(
        _body,
        grid=(nb,),
        in_specs=[
            pl.BlockSpec((1, 1, BR), lambda i: (i, 0, 0)),
            pl.BlockSpec((BR, V), lambda i: (i, 0)),
        ],
        out_specs=[
            pl.BlockSpec((1, 1), lambda i: (0, 0), memory_space=pltpu.SMEM),
            pl.BlockSpec((1, 1), lambda i: (0, 0), memory_space=pltpu.SMEM),
        ],
        out_shape=[
            jax.ShapeDtypeStruct((1, 1), jnp.float32),
            jax.ShapeDtypeStruct((1, 1), jnp.float32),
        ],
    )(t3, p)
    return loss_sum[0, 0] / cnt[0, 0]


# speedup vs baseline: 4.9933x; 1.1070x over previous
import jax
import jax.numpy as jnp
from jax.experimental import pallas as pl
from jax.experimental.pallas import tpu as pltpu

V = 32000
SMOOTHING = 0.1
IGNORE = 0
CONF = 1.0 - SMOOTHING
SMOOTH = SMOOTHING / (V - 1)

BR = 16
W = 256
C = V // W
K = 8


def _acc_reduce(op, slices):
    accs = list(slices[:K])
    for k in range(K, len(slices)):
        accs[k % K] = op(accs[k % K], slices[k])
    while len(accs) > 1:
        nxt = [op(accs[i], accs[i + 1]) for i in range(0, len(accs) - 1, 2)]
        if len(accs) % 2:
            nxt.append(accs[-1])
        accs = nxt
    return accs[0]


def _body(ts_ref, tv_ref, p_ref, loss_ref, cnt_ref):
    i = pl.program_id(0)
    t = tv_ref[0, 0, :]

    xs = [p_ref[:, k * W:(k + 1) * W] for k in range(C)]
    m_l = _acc_reduce(jnp.maximum, xs)
    m = jnp.max(m_l, axis=1, keepdims=True)
    psum = jnp.sum(_acc_reduce(jnp.add, xs), axis=1)

    es = [jnp.exp(p_ref[:, k * W:(k + 1) * W] - m) for k in range(C)]
    s = jnp.sum(_acc_reduce(jnp.add, es), axis=1)

    rows = []
    lane = jax.lax.broadcasted_iota(jnp.int32, (1, 128), 1)
    for r in range(BR):
        tr = ts_ref[0, 0, r]
        off = (tr // 128) * 128
        x = p_ref[pl.ds(r, 1), pl.ds(off, 128)]
        rows.append(jnp.where(lane == (tr - off), x, 0.0))
    pt = jnp.sum(jnp.concatenate(rows, axis=0), axis=1)

    lse = m[:, 0] + jnp.log(s)
    maskf = (t != IGNORE).astype(jnp.float32)
    loss = -(SMOOTH * (psum - V * lse) + (CONF - SMOOTH) * (pt - lse))

    @pl.when(i == 0)
    def _():
        loss_ref[0, 0] = 0.0
        cnt_ref[0, 0] = 0.0

    loss_ref[0, 0] += jnp.sum(loss * maskf)
    cnt_ref[0, 0] += jnp.sum(maskf)


def kernel(pred, target):
    p = pred.reshape(-1, V)
    n = p.shape[0]
    nb = n // BR
    t3 = target.reshape(nb, 1, BR).astype(jnp.int32)

    loss_sum, cnt = pl.pallas_call(
        _body,
        grid=(nb,),
        in_specs=[
            pl.BlockSpec((1, 1, BR), lambda i: (i, 0, 0),
                         memory_space=pltpu.SMEM),
            pl.BlockSpec((1, 1, BR), lambda i: (i, 0, 0)),
            pl.BlockSpec((BR, V), lambda i: (i, 0)),
        ],
        out_specs=[
            pl.BlockSpec((1, 1), lambda i: (0, 0), memory_space=pltpu.SMEM),
            pl.BlockSpec((1, 1), lambda i: (0, 0), memory_space=pltpu.SMEM),
        ],
        out_shape=[
            jax.ShapeDtypeStruct((1, 1), jnp.float32),
            jax.ShapeDtypeStruct((1, 1), jnp.float32),
        ],
    )(t3, t3, p)
    return loss_sum[0, 0] / cnt[0, 0]


# speedup vs baseline: 6.9979x; 1.4015x over previous
import jax
import jax.numpy as jnp
from jax.experimental import pallas as pl
from jax.experimental.pallas import tpu as pltpu

V = 32000
SMOOTHING = 0.1
IGNORE = 0
CONF = 1.0 - SMOOTHING
SMOOTH = SMOOTHING / (V - 1)

BR = 32
W = 256
C = V // W
K = 8


def _acc_reduce(op, slices):
    accs = list(slices[:K])
    for k in range(K, len(slices)):
        accs[k % K] = op(accs[k % K], slices[k])
    while len(accs) > 1:
        nxt = [op(accs[i], accs[i + 1]) for i in range(0, len(accs) - 1, 2)]
        if len(accs) % 2:
            nxt.append(accs[-1])
        accs = nxt
    return accs[0]


def _body(ts_ref, tv_ref, p_ref, loss_ref, cnt_ref):
    i = pl.program_id(0)
    t = tv_ref[0, 0, :]

    xs = [p_ref[:, k * W:(k + 1) * W] for k in range(C)]
    m_l = _acc_reduce(jnp.maximum, xs)
    m = jnp.max(m_l, axis=1, keepdims=True)
    psum = jnp.sum(_acc_reduce(jnp.add, xs), axis=1)

    es = [jnp.exp(p_ref[:, k * W:(k + 1) * W] - m) for k in range(C)]
    s = jnp.sum(_acc_reduce(jnp.add, es), axis=1)

    rows = []
    lane = jax.lax.broadcasted_iota(jnp.int32, (1, 128), 1)
    for r in range(BR):
        tr = ts_ref[0, 0, r]
        off = (tr // 128) * 128
        x = p_ref[pl.ds(r, 1), pl.ds(off, 128)]
        rows.append(jnp.where(lane == (tr - off), x, 0.0))
    pt = jnp.sum(jnp.concatenate(rows, axis=0), axis=1)

    lse = m[:, 0] + jnp.log(s)
    maskf = (t != IGNORE).astype(jnp.float32)
    loss = -(SMOOTH * (psum - V * lse) + (CONF - SMOOTH) * (pt - lse))

    @pl.when(i == 0)
    def _():
        loss_ref[0, 0] = 0.0
        cnt_ref[0, 0] = 0.0

    loss_ref[0, 0] += jnp.sum(loss * maskf)
    cnt_ref[0, 0] += jnp.sum(maskf)


def kernel(pred, target):
    p = pred.reshape(-1, V)
    n = p.shape[0]
    nb = n // BR
    t3 = target.reshape(nb, 1, BR).astype(jnp.int32)

    loss_sum, cnt = pl.pallas_call(
        _body,
        grid=(nb,),
        in_specs=[
            pl.BlockSpec((1, 1, BR), lambda i: (i, 0, 0),
                         memory_space=pltpu.SMEM),
            pl.BlockSpec((1, 1, BR), lambda i: (i, 0, 0)),
            pl.BlockSpec((BR, V), lambda i: (i, 0)),
        ],
        out_specs=[
            pl.BlockSpec((1, 1), lambda i: (0, 0), memory_space=pltpu.SMEM),
            pl.BlockSpec((1, 1), lambda i: (0, 0), memory_space=pltpu.SMEM),
        ],
        out_shape=[
            jax.ShapeDtypeStruct((1, 1), jnp.float32),
            jax.ShapeDtypeStruct((1, 1), jnp.float32),
        ],
    )(t3, t3, p)
    return loss_sum[0, 0] / cnt[0, 0]


# speedup vs baseline: 8.4584x; 1.2087x over previous
import jax
import jax.numpy as jnp
from jax.experimental import pallas as pl
from jax.experimental.pallas import tpu as pltpu

V = 32000
SMOOTHING = 0.1
IGNORE = 0
CONF = 1.0 - SMOOTHING
SMOOTH = SMOOTHING / (V - 1)

BR = 64
W = 256
C = V // W
K = 8


def _acc_reduce(op, slices):
    accs = list(slices[:K])
    for k in range(K, len(slices)):
        accs[k % K] = op(accs[k % K], slices[k])
    while len(accs) > 1:
        nxt = [op(accs[i], accs[i + 1]) for i in range(0, len(accs) - 1, 2)]
        if len(accs) % 2:
            nxt.append(accs[-1])
        accs = nxt
    return accs[0]


def _body(ts_ref, tv_ref, p_ref, loss_ref, cnt_ref):
    i = pl.program_id(0)
    t = tv_ref[0, 0, :]

    xs = [p_ref[:, k * W:(k + 1) * W] for k in range(C)]
    m_l = _acc_reduce(jnp.maximum, xs)
    m = jnp.max(m_l, axis=1, keepdims=True)
    psum = jnp.sum(_acc_reduce(jnp.add, xs), axis=1)

    es = [jnp.exp(p_ref[:, k * W:(k + 1) * W] - m) for k in range(C)]
    s = jnp.sum(_acc_reduce(jnp.add, es), axis=1)

    rows = []
    lane = jax.lax.broadcasted_iota(jnp.int32, (1, 128), 1)
    for r in range(BR):
        tr = ts_ref[0, 0, r]
        off = (tr // 128) * 128
        x = p_ref[pl.ds(r, 1), pl.ds(off, 128)]
        rows.append(jnp.where(lane == (tr - off), x, 0.0))
    pt = jnp.sum(jnp.concatenate(rows, axis=0), axis=1)

    lse = m[:, 0] + jnp.log(s)
    maskf = (t != IGNORE).astype(jnp.float32)
    loss = -(SMOOTH * (psum - V * lse) + (CONF - SMOOTH) * (pt - lse))

    @pl.when(i == 0)
    def _():
        loss_ref[0, 0] = 0.0
        cnt_ref[0, 0] = 0.0

    loss_ref[0, 0] += jnp.sum(loss * maskf)
    cnt_ref[0, 0] += jnp.sum(maskf)


def kernel(pred, target):
    p = pred.reshape(-1, V)
    n = p.shape[0]
    nb = n // BR
    t3 = target.reshape(nb, 1, BR).astype(jnp.int32)

    loss_sum, cnt = pl.pallas_call(
        _body,
        grid=(nb,),
        in_specs=[
            pl.BlockSpec((1, 1, BR), lambda i: (i, 0, 0),
                         memory_space=pltpu.SMEM),
            pl.BlockSpec((1, 1, BR), lambda i: (i, 0, 0)),
            pl.BlockSpec((BR, V), lambda i: (i, 0)),
        ],
        out_specs=[
            pl.BlockSpec((1, 1), lambda i: (0, 0), memory_space=pltpu.SMEM),
            pl.BlockSpec((1, 1), lambda i: (0, 0), memory_space=pltpu.SMEM),
        ],
        out_shape=[
            jax.ShapeDtypeStruct((1, 1), jnp.float32),
            jax.ShapeDtypeStruct((1, 1), jnp.float32),
        ],
    )(t3, t3, p)
    return loss_sum[0, 0] / cnt[0, 0]


# speedup vs baseline: 9.7816x; 1.1564x over previous
import jax
import jax.numpy as jnp
from jax.experimental import pallas as pl
from jax.experimental.pallas import tpu as pltpu

V = 32000
SMOOTHING = 0.1
IGNORE = 0
CONF = 1.0 - SMOOTHING
SMOOTH = SMOOTHING / (V - 1)

BR = 128
W = 256
C = V // W
K = 8


def _acc_reduce(op, slices):
    accs = list(slices[:K])
    for k in range(K, len(slices)):
        accs[k % K] = op(accs[k % K], slices[k])
    while len(accs) > 1:
        nxt = [op(accs[i], accs[i + 1]) for i in range(0, len(accs) - 1, 2)]
        if len(accs) % 2:
            nxt.append(accs[-1])
        accs = nxt
    return accs[0]


def _body(ts_ref, tv_ref, p_ref, loss_ref, cnt_ref):
    i = pl.program_id(0)
    t = tv_ref[0, 0, :]

    xs = [p_ref[:, k * W:(k + 1) * W] for k in range(C)]
    m_l = _acc_reduce(jnp.maximum, xs)
    m = jnp.max(m_l, axis=1, keepdims=True)
    psum = jnp.sum(_acc_reduce(jnp.add, xs), axis=1)

    es = [jnp.exp(p_ref[:, k * W:(k + 1) * W] - m) for k in range(C)]
    s = jnp.sum(_acc_reduce(jnp.add, es), axis=1)

    rows = []
    lane = jax.lax.broadcasted_iota(jnp.int32, (1, 128), 1)
    for r in range(BR):
        tr = ts_ref[0, 0, r]
        off = (tr // 128) * 128
        x = p_ref[pl.ds(r, 1), pl.ds(off, 128)]
        rows.append(jnp.where(lane == (tr - off), x, 0.0))
    pt = jnp.sum(jnp.concatenate(rows, axis=0), axis=1)

    lse = m[:, 0] + jnp.log(s)
    maskf = (t != IGNORE).astype(jnp.float32)
    loss = -(SMOOTH * (psum - V * lse) + (CONF - SMOOTH) * (pt - lse))

    @pl.when(i == 0)
    def _():
        loss_ref[0, 0] = 0.0
        cnt_ref[0, 0] = 0.0

    loss_ref[0, 0] += jnp.sum(loss * maskf)
    cnt_ref[0, 0] += jnp.sum(maskf)


def kernel(pred, target):
    p = pred.reshape(-1, V)
    n = p.shape[0]
    nb = n // BR
    t3 = target.reshape(nb, 1, BR).astype(jnp.int32)

    loss_sum, cnt = pl.pallas_call(
        _body,
        grid=(nb,),
        in_specs=[
            pl.BlockSpec((1, 1, BR), lambda i: (i, 0, 0),
                         memory_space=pltpu.SMEM),
            pl.BlockSpec((1, 1, BR), lambda i: (i, 0, 0)),
            pl.BlockSpec((BR, V), lambda i: (i, 0)),
        ],
        out_specs=[
            pl.BlockSpec((1, 1), lambda i: (0, 0), memory_space=pltpu.SMEM),
            pl.BlockSpec((1, 1), lambda i: (0, 0), memory_space=pltpu.SMEM),
        ],
        out_shape=[
            jax.ShapeDtypeStruct((1, 1), jnp.float32),
            jax.ShapeDtypeStruct((1, 1), jnp.float32),
        ],
    )(t3, t3, p)
    return loss_sum[0, 0] / cnt[0, 0]
